# baseline (speedup 1.0000x reference)
import functools

import jax
import jax.numpy as jnp
from jax import lax
from jax.experimental import pallas as pl
from jax.experimental.pallas import tpu as pltpu
from jax.experimental.pallas import tpu_sc as plsc

P = 7
PQ = P * P
SS = 1.0 / 16.0
H = 50
W = 50
C = 256
NROI = 1000

_info = plsc.get_sparse_core_info()
_NC, _NS = _info.num_cores, _info.num_subcores
NW = _NC * _NS
ROIS_PER_W = (NROI + NW - 1) // NW


def _coords(s, y1v, x1v, bh, bw):
    p = lax.shift_right_logical(s * 9363, 16)
    q = s - p * P
    cy = y1v + (p.astype(jnp.float32) + 0.5) * bh
    cx = x1v + (q.astype(jnp.float32) + 0.5) * bw
    cy = jnp.clip(cy, 0.0, H - 1.0)
    cx = jnp.clip(cx, 0.0, W - 1.0)
    y0 = cy.astype(jnp.int32)
    x0 = cx.astype(jnp.int32)
    wy = cy - y0.astype(jnp.float32)
    wx = cx - x0.astype(jnp.float32)
    yb = jnp.minimum(y0 + 1, H - 1)
    xb = jnp.minimum(x0 + 1, W - 1)
    return y0, x0, yb, xb, wy, wx


def _roi_body(xf_hbm, rois_hbm, out_hbm, rois_v, idx_lo, idx_hi, buf_lo,
              buf_hi, wy_nx, wx_nx, wy_cu, wx_cu, out_v, sem_lo, sem_hi):
    wid = lax.axis_index("s") * _NC + lax.axis_index("c")
    base = wid * ROIS_PER_W
    pltpu.sync_copy(rois_hbm.at[pl.ds(base * 16, ROIS_PER_W * 16)], rois_v)
    iota = lax.iota(jnp.int32, 16)

    def roi_params(i):
        ri = jnp.minimum(i, ROIS_PER_W - 1)
        rr = jnp.minimum(base + ri, NROI - 1) - base
        roi = rois_v[pl.ds(rr * 16, 16)]
        y1v = jnp.full((16,), roi[0]) * SS
        x1v = jnp.full((16,), roi[1]) * SS
        y2v = jnp.full((16,), roi[2]) * SS
        x2v = jnp.full((16,), roi[3]) * SS
        bh = (y2v - y1v) * (1.0 / P)
        bw = (x2v - x1v) * (1.0 / P)
        return y1v, x1v, bh, bw

    def build_lo(i):
        y1v, x1v, bh, bw = roi_params(i)
        for grp in range(4):
            s = iota + grp * 16
            y0, x0, yb, xb, wy, wx = _coords(s, y1v, x1v, bh, bw)
            idx_lo[pl.ds(grp * 16, 16)] = y0 * W + x0
            idx_lo[pl.ds(64 + grp * 16, 16)] = y0 * W + xb
            wy_nx[pl.ds(grp * 16, 16)] = wy
            wx_nx[pl.ds(grp * 16, 16)] = wx
        pltpu.async_copy(xf_hbm.at[idx_lo], buf_lo, sem_lo)

    def build_hi(i):
        y1v, x1v, bh, bw = roi_params(i)
        for grp in range(4):
            s = iota + grp * 16
            y0, x0, yb, xb, wy, wx = _coords(s, y1v, x1v, bh, bw)
            idx_hi[pl.ds(grp * 16, 16)] = yb * W + x0
            idx_hi[pl.ds(64 + grp * 16, 16)] = yb * W + xb
        pltpu.async_copy(xf_hbm.at[idx_hi], buf_hi, sem_hi)

    def wait_lo():
        pltpu.make_async_copy(xf_hbm.at[idx_lo], buf_lo, sem_lo).wait()

    def wait_hi():
        pltpu.make_async_copy(xf_hbm.at[idx_hi], buf_hi, sem_hi).wait()

    build_lo(0)
    build_hi(0)

    def roi_step(i, carry):
        for grp in range(4):
            wy_cu[pl.ds(grp * 16, 16)] = wy_nx[pl.ds(grp * 16, 16)]
            wx_cu[pl.ds(grp * 16, 16)] = wx_nx[pl.ds(grp * 16, 16)]

        wait_lo()

        def pass1(pq, inner):
            wxv = jnp.full((16,), wx_cu[pl.ds(pq, 16)][0])
            for g in range(C // 16):
                g00 = buf_lo[pq, pl.ds(g * 16, 16)]
                g01 = buf_lo[pq + 64, pl.ds(g * 16, 16)]
                out_v[pl.ds(pq * C + g * 16, 16)] = g00 + wxv * (g01 - g00)
            return inner

        lax.fori_loop(0, PQ, pass1, 0)
        build_lo(i + 1)
        wait_hi()

        def pass2(pq, inner):
            wxv = jnp.full((16,), wx_cu[pl.ds(pq, 16)][0])
            wyv = jnp.full((16,), wy_cu[pl.ds(pq, 16)][0])
            for g in range(C // 16):
                g10 = buf_hi[pq, pl.ds(g * 16, 16)]
                g11 = buf_hi[pq + 64, pl.ds(g * 16, 16)]
                a1 = g10 + wxv * (g11 - g10)
                a0 = out_v[pl.ds(pq * C + g * 16, 16)]
                out_v[pl.ds(pq * C + g * 16, 16)] = a0 + wyv * (a1 - a0)
            return inner

        lax.fori_loop(0, PQ, pass2, 0)
        build_hi(i + 1)
        r = jnp.minimum(base + i, NROI - 1)
        pltpu.sync_copy(out_v, out_hbm.at[r])
        return carry

    lax.fori_loop(0, ROIS_PER_W, roi_step, 0)
    wait_lo()
    wait_hi()


@functools.partial(
    pl.kernel,
    out_type=jax.ShapeDtypeStruct((NROI, PQ * C), jnp.float32),
    mesh=plsc.VectorSubcoreMesh(core_axis_name="c", subcore_axis_name="s"),
    scratch_types=[
        pltpu.VMEM((16 * ROIS_PER_W,), jnp.float32),
        pltpu.VMEM((128,), jnp.int32),
        pltpu.VMEM((128,), jnp.int32),
        pltpu.VMEM((128, C), jnp.float32),
        pltpu.VMEM((128, C), jnp.float32),
        pltpu.VMEM((64,), jnp.float32),
        pltpu.VMEM((64,), jnp.float32),
        pltpu.VMEM((64,), jnp.float32),
        pltpu.VMEM((64,), jnp.float32),
        pltpu.VMEM((PQ * C,), jnp.float32),
        pltpu.SemaphoreType.DMA,
        pltpu.SemaphoreType.DMA,
    ],
)
def _roi_pool_sc(xf_hbm, rois_hbm, out_hbm, *rest):
    _roi_body(xf_hbm, rois_hbm, out_hbm, *rest)


TR_BLK = 8


def _tr_body(in_ref, out_ref):
    out_ref[...] = jnp.swapaxes(in_ref[...], 1, 2)


_transpose_tc = pl.pallas_call(
    _tr_body,
    grid=(NROI // TR_BLK,),
    in_specs=[pl.BlockSpec((TR_BLK, PQ, C), lambda i: (i, 0, 0))],
    out_specs=pl.BlockSpec((TR_BLK, C, PQ), lambda i: (i, 0, 0)),
    out_shape=jax.ShapeDtypeStruct((NROI, C, PQ), jnp.float32),
)


def kernel(x, rois, roi_indices):
    b, c, h, w = x.shape
    xf = jnp.transpose(x, (0, 2, 3, 1)).reshape(b * h * w, c)
    rois16 = jnp.pad(rois.astype(jnp.float32), ((0, NW * ROIS_PER_W - NROI),
                                                (0, 12))).reshape(-1)
    pooled = _roi_pool_sc(xf, rois16)
    out = _transpose_tc(pooled.reshape(NROI, PQ, C))
    return out.reshape(NROI, C * PQ)


# speedup vs baseline: 1.5541x; 1.5541x over previous
import functools

import jax
import jax.numpy as jnp
from jax import lax
from jax.experimental import pallas as pl
from jax.experimental.pallas import tpu as pltpu
from jax.experimental.pallas import tpu_sc as plsc

P = 7
PQ = P * P
SS = 1.0 / 16.0
H = 50
W = 50
C = 256
NROI = 1000

_info = plsc.get_sparse_core_info()
_NC, _NS = _info.num_cores, _info.num_subcores
NW = _NC * _NS
ROIS_PER_W = (NROI + NW - 1) // NW


def _coords(s, y1v, x1v, bh, bw):
    p = lax.shift_right_logical(s * 9363, 16)
    q = s - p * P
    cy = y1v + (p.astype(jnp.float32) + 0.5) * bh
    cx = x1v + (q.astype(jnp.float32) + 0.5) * bw
    cy = jnp.clip(cy, 0.0, H - 1.0)
    cx = jnp.clip(cx, 0.0, W - 1.0)
    y0 = cy.astype(jnp.int32)
    x0 = cx.astype(jnp.int32)
    wy = cy - y0.astype(jnp.float32)
    wx = cx - x0.astype(jnp.float32)
    yb = jnp.minimum(y0 + 1, H - 1)
    xb = jnp.minimum(x0 + 1, W - 1)
    return y0, x0, yb, xb, wy, wx


def _roi_body(xf_hbm, rois_hbm, out_hbm, rois_v, idx_lo, idx_hi, buf_lo,
              buf_hi, wy_nx, wx_nx, wy_cu, wx_cu, out_v, sem_lo, sem_hi):
    wid = lax.axis_index("s") * _NC + lax.axis_index("c")
    base = wid * ROIS_PER_W
    pltpu.sync_copy(rois_hbm.at[pl.ds(base * 16, ROIS_PER_W * 16)], rois_v)
    iota = lax.iota(jnp.int32, 16)

    def roi_params(i):
        ri = jnp.minimum(i, ROIS_PER_W - 1)
        rr = jnp.minimum(base + ri, NROI - 1) - base
        roi = rois_v[pl.ds(rr * 16, 16)]
        y1v = jnp.full((16,), roi[0]) * SS
        x1v = jnp.full((16,), roi[1]) * SS
        y2v = jnp.full((16,), roi[2]) * SS
        x2v = jnp.full((16,), roi[3]) * SS
        bh = (y2v - y1v) * (1.0 / P)
        bw = (x2v - x1v) * (1.0 / P)
        return y1v, x1v, bh, bw

    def build_lo(i):
        y1v, x1v, bh, bw = roi_params(i)
        for grp in range(4):
            s = iota + grp * 16
            y0, x0, yb, xb, wy, wx = _coords(s, y1v, x1v, bh, bw)
            idx_lo[pl.ds(grp * 16, 16)] = y0 * W + x0
            idx_lo[pl.ds(64 + grp * 16, 16)] = y0 * W + xb
            wy_nx[pl.ds(grp * 16, 16)] = wy
            wx_nx[pl.ds(grp * 16, 16)] = wx
        pltpu.async_copy(xf_hbm.at[idx_lo], buf_lo, sem_lo)

    def build_hi(i):
        y1v, x1v, bh, bw = roi_params(i)
        for grp in range(4):
            s = iota + grp * 16
            y0, x0, yb, xb, wy, wx = _coords(s, y1v, x1v, bh, bw)
            idx_hi[pl.ds(grp * 16, 16)] = yb * W + x0
            idx_hi[pl.ds(64 + grp * 16, 16)] = yb * W + xb
        pltpu.async_copy(xf_hbm.at[idx_hi], buf_hi, sem_hi)

    def wait_lo():
        pltpu.make_async_copy(xf_hbm.at[idx_lo], buf_lo, sem_lo).wait()

    def wait_hi():
        pltpu.make_async_copy(xf_hbm.at[idx_hi], buf_hi, sem_hi).wait()

    build_lo(0)
    build_hi(0)

    def roi_step(i, carry):
        for grp in range(4):
            wy_cu[pl.ds(grp * 16, 16)] = wy_nx[pl.ds(grp * 16, 16)]
            wx_cu[pl.ds(grp * 16, 16)] = wx_nx[pl.ds(grp * 16, 16)]

        wait_lo()

        @plsc.parallel_loop(0, PQ, unroll=2)
        def pass1(pq):
            wxv = jnp.full((16,), wx_cu[pl.ds(pq, 16)][0])
            for g in range(C // 16):
                g00 = buf_lo[pq, pl.ds(g * 16, 16)]
                g01 = buf_lo[pq + 64, pl.ds(g * 16, 16)]
                out_v[pl.ds(pq * C + g * 16, 16)] = g00 + wxv * (g01 - g00)
        build_lo(i + 1)
        wait_hi()

        @plsc.parallel_loop(0, PQ, unroll=2)
        def pass2(pq):
            wxv = jnp.full((16,), wx_cu[pl.ds(pq, 16)][0])
            wyv = jnp.full((16,), wy_cu[pl.ds(pq, 16)][0])
            for g in range(C // 16):
                g10 = buf_hi[pq, pl.ds(g * 16, 16)]
                g11 = buf_hi[pq + 64, pl.ds(g * 16, 16)]
                a1 = g10 + wxv * (g11 - g10)
                a0 = out_v[pl.ds(pq * C + g * 16, 16)]
                out_v[pl.ds(pq * C + g * 16, 16)] = a0 + wyv * (a1 - a0)
        build_hi(i + 1)
        r = jnp.minimum(base + i, NROI - 1)
        pltpu.sync_copy(out_v, out_hbm.at[r])
        return carry

    lax.fori_loop(0, ROIS_PER_W, roi_step, 0)
    wait_lo()
    wait_hi()


@functools.partial(
    pl.kernel,
    out_type=jax.ShapeDtypeStruct((NROI, PQ * C), jnp.float32),
    mesh=plsc.VectorSubcoreMesh(core_axis_name="c", subcore_axis_name="s"),
    scratch_types=[
        pltpu.VMEM((16 * ROIS_PER_W,), jnp.float32),
        pltpu.VMEM((128,), jnp.int32),
        pltpu.VMEM((128,), jnp.int32),
        pltpu.VMEM((128, C), jnp.float32),
        pltpu.VMEM((128, C), jnp.float32),
        pltpu.VMEM((64,), jnp.float32),
        pltpu.VMEM((64,), jnp.float32),
        pltpu.VMEM((64,), jnp.float32),
        pltpu.VMEM((64,), jnp.float32),
        pltpu.VMEM((PQ * C,), jnp.float32),
        pltpu.SemaphoreType.DMA,
        pltpu.SemaphoreType.DMA,
    ],
)
def _roi_pool_sc(xf_hbm, rois_hbm, out_hbm, *rest):
    _roi_body(xf_hbm, rois_hbm, out_hbm, *rest)


TR_BLK = 8


def _tr_body(in_ref, out_ref):
    out_ref[...] = jnp.swapaxes(in_ref[...], 1, 2)


_transpose_tc = pl.pallas_call(
    _tr_body,
    grid=(NROI // TR_BLK,),
    in_specs=[pl.BlockSpec((TR_BLK, PQ, C), lambda i: (i, 0, 0))],
    out_specs=pl.BlockSpec((TR_BLK, C, PQ), lambda i: (i, 0, 0)),
    out_shape=jax.ShapeDtypeStruct((NROI, C, PQ), jnp.float32),
)


def kernel(x, rois, roi_indices):
    b, c, h, w = x.shape
    xf = jnp.transpose(x, (0, 2, 3, 1)).reshape(b * h * w, c)
    rois16 = jnp.pad(rois.astype(jnp.float32), ((0, NW * ROIS_PER_W - NROI),
                                                (0, 12))).reshape(-1)
    pooled = _roi_pool_sc(xf, rois16)
    out = _transpose_tc(pooled.reshape(NROI, PQ, C))
    return out.reshape(NROI, C * PQ)
